# baseline (speedup 1.0000x reference)
import functools

import jax
import jax.numpy as jnp
from jax import lax
from jax.experimental import pallas as pl

def _keys_of(x):
    b = lax.bitcast_convert_type(x, jnp.int32)
    return jnp.where(b >= 0, b, jnp.bitwise_not(jnp.bitwise_and(b, jnp.int32(2147483647))))


def _float_of(k):
    b = jnp.where(k >= 0, k, jnp.bitwise_or(jnp.bitwise_not(k), jnp.int32(-2147483648)))
    return lax.bitcast_convert_type(b, jnp.float32)


def _row_stats(x):
    R, n = x.shape
    inv_n = jnp.float32(1.0 / n)
    mean = jnp.sum(x, axis=1, keepdims=True) * inv_n
    var = jnp.sum((x - mean) ** 2, axis=1, keepdims=True) * jnp.float32(
        1.0 / (n - 1))
    mn = jnp.min(x, axis=1, keepdims=True)
    mx = jnp.max(x, axis=1, keepdims=True)

    key = _keys_of(x)
    ks = (n // 4, n // 2, (3 * n) // 4)

    hi16 = lax.shift_right_arithmetic(key, 16).astype(jnp.int16)
    lo16 = jnp.bitwise_xor(jnp.bitwise_and(key, jnp.int32(0xFFFF)),
                           jnp.int32(0x8000)).astype(jnp.int16)

    def phase1(i, carry):
        vs, cs = carry
        shift = jnp.left_shift(jnp.int32(1), jnp.int32(15) - i)
        nvs, ncs = [], []
        for j in range(3):
            t = vs[j] + shift
            cnt = jnp.sum((hi16 < t.astype(jnp.int16)).astype(jnp.int32),
                          axis=1, keepdims=True)
            acc = cnt <= ks[j]
            nvs.append(jnp.where(acc, t, vs[j]))
            ncs.append(jnp.where(acc, cnt, cs[j]))
        return tuple(nvs), tuple(ncs)

    v0 = jnp.full((R, 1), -32768, dtype=jnp.int32)
    c0 = jnp.zeros((R, 1), dtype=jnp.int32)
    (ps, cbases) = lax.fori_loop(0, 16, phase1, ((v0, v0, v0), (c0, c0, c0)))

    ms = [jnp.where(hi16 == ps[j].astype(jnp.int16), lo16, jnp.int16(32767))
          for j in range(3)]

    def phase2(i, carry):
        vs, cs = carry
        shift = jnp.left_shift(jnp.int32(1), jnp.int32(15) - i)
        nvs, ncs = [], []
        for j in range(3):
            t = vs[j] + shift
            cnt = jnp.sum((ms[j] < t.astype(jnp.int16)).astype(jnp.int32),
                          axis=1, keepdims=True)
            acc = (cbases[j] + cnt) <= ks[j]
            nvs.append(jnp.where(acc, t, vs[j]))
            ncs.append(jnp.where(acc, cnt, cs[j]))
        return tuple(nvs), tuple(ncs)

    (ls, c2s) = lax.fori_loop(0, 16, phase2, ((v0, v0, v0), (c0, c0, c0)))

    qs = []
    fracs = (0.75, 0.5, 0.25)
    for j in range(3):
        v = jnp.bitwise_or(
            jnp.left_shift(ps[j], 16),
            jnp.bitwise_xor(jnp.bitwise_and(ls[j], jnp.int32(0xFFFF)),
                            jnp.int32(0x8000)))
        cfin = cbases[j] + c2s[j]
        below = key < v
        lo_key = jnp.max(jnp.where(below, key, jnp.int32(-2147483648)),
                         axis=1, keepdims=True)
        lo_key = jnp.where(cfin <= ks[j] - 1, v, lo_key)
        hi = _float_of(v)
        lo = _float_of(lo_key)
        f = jnp.float32(fracs[j])
        qs.append((1.0 - f) * lo + f * hi)

    return jnp.concatenate([mean, var, mn, qs[0], qs[1], qs[2], mx], axis=1)


def _fused_kernel(w0, b0, w1, b1, w2, b2, w1t, w2t, bias1, bias2, out):
    feats = []
    for p in (w0, b0, w1, b1, w2, b2):
        feats.append(_row_stats(p[...]))
    feats.append(jnp.zeros((feats[0].shape[0], 6), dtype=jnp.float32))
    f = jnp.concatenate(feats, axis=1)
    h = jnp.dot(f, w1t[...], preferred_element_type=jnp.float32)
    h = jnp.maximum(h + bias1[...], 0.0)
    out[...] = jnp.dot(h, w2t[...],
                       preferred_element_type=jnp.float32) + bias2[...]


def kernel(w0, b0, w1, b1, w2, b2, mlp_w1, mlp_b1, mlp_w2, mlp_b2):
    B = w0.shape[0]
    R = 128
    grid = (B // R,)

    w0f = w0.reshape(B, -1)
    w1f = w1.reshape(B, -1)
    w2f = w2.reshape(B, -1)

    w1t = jnp.pad(mlp_w1, ((0, 0), (0, 6))).T
    w2t = mlp_w2.T
    bias1 = mlp_b1.reshape(1, -1)
    bias2 = mlp_b2.reshape(1, -1)

    H = mlp_w2.shape[0]
    nw = w0f.shape[1]
    nb = b0.shape[-1]

    row_spec_w = pl.BlockSpec((R, nw), lambda i: (i, 0))
    row_spec_b = pl.BlockSpec((R, nb), lambda i: (i, 0))
    full = lambda a: pl.BlockSpec(a.shape, lambda i: tuple(0 for _ in a.shape))

    return pl.pallas_call(
        _fused_kernel,
        grid=grid,
        in_specs=[
            row_spec_w, row_spec_b, row_spec_w, row_spec_b, row_spec_w,
            row_spec_b, full(w1t), full(w2t), full(bias1), full(bias2),
        ],
        out_specs=pl.BlockSpec((R, H), lambda i: (i, 0)),
        out_shape=jax.ShapeDtypeStruct((B, H), jnp.float32),
    )(w0f, b0, w1f, b1, w2f, b2, w1t, w2t, bias1, bias2)
